# baseline (speedup 1.0000x reference)
import functools

import jax
import jax.numpy as jnp
from jax import lax
from jax.experimental import pallas as pl
from jax.experimental.pallas import tpu as pltpu
from jax.experimental.pallas import tpu_sc as plsc

E = 320000
R = 10000
RP = 10240
D = 128
NC = 2
NS = 16
NWORK = NC * NS
EPT = E // NWORK
W = 128
NFULL = EPT // W
NWINP = NFULL + 2
TAIL = EPT - W
RPT = RP // NS
LANES = 16
ZR = 16
NB = 2
NGRP = NWINP // NB


def _win_off(j):
    return jnp.where(
        j == NFULL, TAIL, jnp.where(j == NFULL + 1, 0, j * W)
    )


def _sc_scatter_body(
    i1_hbm, idx_hbm, out_hbm, idx_v, upd_v, zrow_v, acc_sh, isem, gsems, ssems
):
    c = lax.axis_index("c")
    s = lax.axis_index("s")
    wid = c * NS + s
    ebase = wid * EPT

    idx_cp = pltpu.async_copy(idx_hbm.at[wid], idx_v, isem)
    prime = [
        pltpu.async_copy(
            i1_hbm.at[pl.ds(ebase + b * W, W)], upd_v.at[b], gsems[b]
        )
        for b in range(NB)
    ]

    def zero_row(i, carry):
        for blk in range(D // LANES):
            zrow_v[i, pl.ds(blk * LANES, LANES)] = jnp.zeros((LANES,), jnp.float32)
        return carry

    lax.fori_loop(0, ZR, zero_row, 0)
    for r in range(RPT // ZR):
        pltpu.sync_copy(zrow_v, acc_sh.at[pl.ds(s * RPT + r * ZR, ZR)])
    idx_cp.wait()
    plsc.subcore_barrier()

    def group(g, carry):
        base = g * NB
        scats = []
        for b in range(NB):
            prime[b].wait()
            scats.append(
                pltpu.async_copy(
                    upd_v.at[b], acc_sh.at[idx_v.at[base + b]], ssems[b], add=True
                )
            )
        for b in range(NB):
            scats[b].wait()
            off = _win_off(base + b + NB)
            pltpu.async_copy(
                i1_hbm.at[pl.ds(ebase + off, W)], upd_v.at[b], gsems[b]
            )
        return carry

    lax.fori_loop(0, NGRP - 1, group, 0)
    base = (NGRP - 1) * NB
    scats = []
    for b in range(NB):
        prime[b].wait()
        scats.append(
            pltpu.async_copy(
                upd_v.at[b], acc_sh.at[idx_v.at[base + b]], ssems[b], add=True
            )
        )
    for b in range(NB):
        scats[b].wait()
    plsc.subcore_barrier()

    rbase = s * RPT
    pltpu.sync_copy(
        acc_sh.at[pl.ds(rbase, RPT)],
        out_hbm.at[c, pl.ds(rbase, RPT)],
    )


_sc_scatter = functools.partial(
    pl.kernel,
    out_type=jax.ShapeDtypeStruct((NC, RP, D), jnp.float32),
    mesh=plsc.VectorSubcoreMesh(
        core_axis_name="c", subcore_axis_name="s", num_cores=NC, num_subcores=NS
    ),
    scratch_types=[
        pltpu.VMEM((NWINP, W), jnp.int32),
        pltpu.VMEM((NB, W, D), jnp.float32),
        pltpu.VMEM((ZR, D), jnp.float32),
        pltpu.VMEM_SHARED((RP, D), jnp.float32),
        pltpu.SemaphoreType.DMA,
        [pltpu.SemaphoreType.DMA] * NB,
        [pltpu.SemaphoreType.DMA] * NB,
    ],
)(_sc_scatter_body)


def _sum_partials_body(a_ref, b_ref, o_ref):
    o_ref[...] = a_ref[0] + b_ref[0]


def kernel(i1, pair_i, p1):
    del p1
    pi = pair_i.astype(jnp.int32).reshape(NWORK, EPT)
    trash = R + (jnp.arange(NWORK * 2 * W, dtype=jnp.int32) % (RP - R)).reshape(
        NWORK, 2 * W
    )
    main = pi[:, : NFULL * W].reshape(NWORK, NFULL, W)
    ntrash78 = W - (EPT - NFULL * W)
    w78 = jnp.concatenate(
        [trash[:, :ntrash78], pi[:, NFULL * W :]], axis=1
    )
    w79 = trash[:, W:]
    idx = jnp.concatenate(
        [main, w78[:, None, :], w79[:, None, :]], axis=1
    )
    partials = _sc_scatter(i1, idx)
    blk = 1000
    out = pl.pallas_call(
        _sum_partials_body,
        out_shape=jax.ShapeDtypeStruct((R, D), jnp.float32),
        grid=(R // blk,),
        in_specs=[
            pl.BlockSpec((1, blk, D), lambda i: (0, i, 0)),
            pl.BlockSpec((1, blk, D), lambda i: (1, i, 0)),
        ],
        out_specs=pl.BlockSpec((blk, D), lambda i: (i, 0)),
    )(partials, partials)
    return out


# speedup vs baseline: 1.2173x; 1.2173x over previous
import functools

import jax
import jax.numpy as jnp
from jax import lax
from jax.experimental import pallas as pl
from jax.experimental.pallas import tpu as pltpu
from jax.experimental.pallas import tpu_sc as plsc

E = 320000
R = 10000
RP = 10240
D = 128
NC = 2
NS = 16
NWORK = NC * NS
EPT = E // NWORK
W = 80
NB = 2
NFULL = EPT // W
REM = EPT - NFULL * W
NWIN = NFULL + (1 if REM else 0)
NWINP = -(-NWIN // NB) * NB
TAIL = EPT - W
RPT = RP // NS
LANES = 16
ZR = 16
NGRP = NWINP // NB


def _win_off(j):
    off = jnp.where(j < NFULL, j * W, TAIL if REM else 0)
    if REM:
        off = jnp.where(j > NFULL, 0, off)
    return off


def _sc_scatter_body(
    i1_hbm, idx_hbm, out_hbm, idx_v, upd_v, zrow_v, acc_sh, isem, gsems
):
    c = lax.axis_index("c")
    s = lax.axis_index("s")
    wid = c * NS + s
    ebase = wid * EPT

    idx_cp = pltpu.async_copy(idx_hbm.at[wid], idx_v, isem)
    prime = [
        pltpu.async_copy(
            i1_hbm.at[pl.ds(ebase + b * W, W)], upd_v.at[b], gsems[b]
        )
        for b in range(NB)
    ]

    def zero_row(i, carry):
        for blk in range(D // LANES):
            zrow_v[i, pl.ds(blk * LANES, LANES)] = jnp.zeros((LANES,), jnp.float32)
        return carry

    lax.fori_loop(0, ZR, zero_row, 0)
    for r in range(RPT // ZR):
        pltpu.sync_copy(zrow_v, acc_sh.at[pl.ds(s * RPT + r * ZR, ZR)])
    idx_cp.wait()
    plsc.subcore_barrier()

    def group(g, carry):
        for b in range(NB):
            j = g * NB + b
            prime[b].wait()
            pltpu.sync_copy(upd_v.at[b], acc_sh.at[idx_v.at[j]], add=True)
            off = _win_off(j + NB)
            pltpu.async_copy(
                i1_hbm.at[pl.ds(ebase + off, W)], upd_v.at[b], gsems[b]
            )
        return carry

    lax.fori_loop(0, NGRP - 1, group, 0)
    base = (NGRP - 1) * NB
    for b in range(NB):
        prime[b].wait()
        pltpu.sync_copy(upd_v.at[b], acc_sh.at[idx_v.at[base + b]], add=True)
    plsc.subcore_barrier()

    rbase = s * RPT
    pltpu.sync_copy(
        acc_sh.at[pl.ds(rbase, RPT)],
        out_hbm.at[c, pl.ds(rbase, RPT)],
    )


_sc_scatter = functools.partial(
    pl.kernel,
    out_type=jax.ShapeDtypeStruct((NC, RP, D), jnp.float32),
    mesh=plsc.VectorSubcoreMesh(
        core_axis_name="c", subcore_axis_name="s", num_cores=NC, num_subcores=NS
    ),
    scratch_types=[
        pltpu.VMEM((NWINP, W), jnp.int32),
        pltpu.VMEM((NB, W, D), jnp.float32),
        pltpu.VMEM((ZR, D), jnp.float32),
        pltpu.VMEM_SHARED((RP, D), jnp.float32),
        pltpu.SemaphoreType.DMA,
        [pltpu.SemaphoreType.DMA] * NB,
    ],
)(_sc_scatter_body)


def _sum_partials_body(a_ref, b_ref, o_ref):
    o_ref[...] = a_ref[0] + b_ref[0]


def kernel(i1, pair_i, p1):
    del p1
    pi = pair_i.astype(jnp.int32).reshape(NWORK, EPT)
    ntrash = (W - REM if REM else 0) + (NWINP - NWIN) * W
    parts = [pi[:, : NFULL * W].reshape(NWORK, NFULL, W)]
    if ntrash:
        trash = R + (
            jnp.arange(NWORK * ntrash, dtype=jnp.int32) % (RP - R)
        ).reshape(NWORK, ntrash)
        rest = jnp.concatenate([trash[:, : W - REM], pi[:, NFULL * W :]], axis=1) \
            if REM else trash[:, :0]
        rest = jnp.concatenate(
            [rest, trash[:, W - REM if REM else 0 :]], axis=1
        ).reshape(NWORK, NWINP - NFULL, W)
        parts.append(rest)
    idx = jnp.concatenate(parts, axis=1) if len(parts) > 1 else parts[0]
    partials = _sc_scatter(i1, idx)
    blk = 1000
    out = pl.pallas_call(
        _sum_partials_body,
        out_shape=jax.ShapeDtypeStruct((R, D), jnp.float32),
        grid=(R // blk,),
        in_specs=[
            pl.BlockSpec((1, blk, D), lambda i: (0, i, 0)),
            pl.BlockSpec((1, blk, D), lambda i: (1, i, 0)),
        ],
        out_specs=pl.BlockSpec((blk, D), lambda i: (i, 0)),
    )(partials, partials)
    return out
